# Initial kernel scaffold; baseline (speedup 1.0000x reference)
import functools

import jax
import jax.numpy as jnp
from jax import lax
from jax.experimental import pallas as pl
from jax.experimental.pallas import tpu as pltpu
from jax.experimental.pallas import tpu_sc as plsc

B, S, C, D = 1024, 200, 16, 16
N = B * S * C

_info = plsc.get_sparse_core_info()
NC, NS = _info.num_cores, _info.num_subcores
NW = NC * NS
PER_W = N // NW
CHUNK = 2048
NCHUNK = PER_W // CHUNK

_mesh = plsc.VectorSubcoreMesh(core_axis_name="c", subcore_axis_name="s")


@functools.partial(
    pl.kernel,
    mesh=_mesh,
    out_type=jax.ShapeDtypeStruct((N, D), jnp.float32),
    scratch_types=[
        pltpu.VMEM((CHUNK,), jnp.int32),
        pltpu.VMEM((CHUNK, D), jnp.float32),
        pltpu.SemaphoreType.DMA,
    ],
)
def _emb_lookup(idx_hbm, table_hbm, out_hbm, idx_v, rows_v, sem):
    wid = lax.axis_index("s") * NC + lax.axis_index("c")
    base = wid * PER_W

    def body(i, carry):
        off = base + i * CHUNK
        pltpu.sync_copy(idx_hbm.at[pl.ds(off, CHUNK)], idx_v)
        pltpu.async_copy(table_hbm.at[idx_v], rows_v, sem).wait()
        pltpu.sync_copy(rows_v, out_hbm.at[pl.ds(off, CHUNK)])
        return carry

    lax.fori_loop(0, NCHUNK, body, 0)


def kernel(inputs, table):
    flat = inputs.reshape(N)
    out = _emb_lookup(flat, table)
    return out.reshape(B, S, C, D)


# speedup vs baseline: 5.7241x; 5.7241x over previous
import functools

import jax
import jax.numpy as jnp
from jax import lax
from jax.experimental import pallas as pl
from jax.experimental.pallas import tpu as pltpu
from jax.experimental.pallas import tpu_sc as plsc

B, S, C, D = 1024, 200, 16, 16
N = B * S * C

_info = plsc.get_sparse_core_info()
NC, NS = _info.num_cores, _info.num_subcores
NW = NC * NS
PER_W = N // NW
CHUNK = 2048
NCHUNK = PER_W // CHUNK

_mesh = plsc.VectorSubcoreMesh(core_axis_name="c", subcore_axis_name="s")


@functools.partial(
    pl.kernel,
    mesh=_mesh,
    out_type=jax.ShapeDtypeStruct((N, D), jnp.float32),
    scratch_types=[
        pltpu.VMEM((CHUNK,), jnp.int32),
        pltpu.VMEM((CHUNK, D), jnp.float32),
        pltpu.SemaphoreType.DMA,
    ],
    compiler_params=pltpu.CompilerParams(use_tc_tiling_on_sc=False),
)
def _emb_lookup(idx_hbm, table_hbm, out_hbm, idx_v, rows_v, sem):
    wid = lax.axis_index("s") * NC + lax.axis_index("c")
    base = wid * PER_W

    def body(i, carry):
        off = base + i * CHUNK
        pltpu.sync_copy(idx_hbm.at[pl.ds(off, CHUNK)], idx_v)
        pltpu.async_copy(table_hbm.at[idx_v], rows_v, sem).wait()
        pltpu.sync_copy(rows_v, out_hbm.at[pl.ds(off, CHUNK)])
        return carry

    lax.fori_loop(0, NCHUNK, body, 0)


def kernel(inputs, table):
    flat = inputs.reshape(N)
    out = _emb_lookup(flat, table)
    return out.reshape(B, S, C, D)


# speedup vs baseline: 5.7497x; 1.0045x over previous
import functools

import jax
import jax.numpy as jnp
from jax import lax
from jax.experimental import pallas as pl
from jax.experimental.pallas import tpu as pltpu
from jax.experimental.pallas import tpu_sc as plsc

B, S, C, D = 1024, 200, 16, 16
N = B * S * C

_info = plsc.get_sparse_core_info()
NC, NS = _info.num_cores, _info.num_subcores
NW = NC * NS
PER_W = N // NW
CHUNK = 1024
NCHUNK = PER_W // CHUNK
NB = 4
NSTEADY = NCHUNK // NB - 1

_mesh = plsc.VectorSubcoreMesh(core_axis_name="c", subcore_axis_name="s")


@functools.partial(
    pl.kernel,
    mesh=_mesh,
    out_type=jax.ShapeDtypeStruct((N, D), jnp.float32),
    scratch_types=(
        [pltpu.VMEM((NB, CHUNK), jnp.int32), pltpu.VMEM((NB, CHUNK, D), jnp.float32)]
        + [pltpu.SemaphoreType.DMA] * (3 * NB)
    ),
    compiler_params=pltpu.CompilerParams(use_tc_tiling_on_sc=False),
)
def _emb_lookup(idx_hbm, table_hbm, out_hbm, idx_v, rows_v, *sems):
    sem_i, sem_g, sem_o = sems[:NB], sems[NB : 2 * NB], sems[2 * NB :]
    wid = lax.axis_index("s") * NC + lax.axis_index("c")
    base = wid * PER_W

    def idx_copy(i, b):
        return pltpu.make_async_copy(
            idx_hbm.at[pl.ds(base + i * CHUNK, CHUNK)], idx_v.at[b], sem_i[b]
        )

    def gather_copy(b):
        return pltpu.make_async_copy(table_hbm.at[idx_v.at[b]], rows_v.at[b], sem_g[b])

    def out_copy(i, b):
        return pltpu.make_async_copy(
            rows_v.at[b], out_hbm.at[pl.ds(base + i * CHUNK, CHUNK)], sem_o[b]
        )

    for b in range(NB):
        idx_copy(b, b).start()
    for b in range(NB):
        idx_copy(b, b).wait()
        gather_copy(b).start()

    def body(k, carry):
        for b in range(NB):
            i = (k + 1) * NB + b
            gather_copy(b).wait()
            out_copy(i - NB, b).start()
            idx_copy(i, b).start()
            idx_copy(i, b).wait()
            out_copy(i - NB, b).wait()
            gather_copy(b).start()
        return carry

    lax.fori_loop(0, NSTEADY, body, 0)

    for b in range(NB):
        i = NCHUNK - NB + b
        gather_copy(b).wait()
        out_copy(i, b).start()
    for b in range(NB):
        out_copy(NCHUNK - NB + b, b).wait()


def kernel(inputs, table):
    flat = inputs.reshape(N)
    out = _emb_lookup(flat, table)
    return out.reshape(B, S, C, D)


# speedup vs baseline: 14.2547x; 2.4792x over previous
import functools

import jax
import jax.numpy as jnp
from jax import lax
from jax.experimental import pallas as pl
from jax.experimental.pallas import tpu as pltpu
from jax.experimental.pallas import tpu_sc as plsc

B, S, C, D = 1024, 200, 16, 16
V = 1000
NP = S * C
N = NP * B

_info = plsc.get_sparse_core_info()
NC, NS, L = _info.num_cores, _info.num_subcores, _info.num_lanes
NW = NC * NS
PPW = NP // NW
NB = 2
NGRP = B // L

_mesh = plsc.VectorSubcoreMesh(core_axis_name="c", subcore_axis_name="s")


@functools.partial(
    pl.kernel,
    mesh=_mesh,
    out_type=jax.ShapeDtypeStruct((S, C, D, B), jnp.float32),
    scratch_types=(
        [pltpu.VMEM((V * D,), jnp.float32)]
        + [pltpu.VMEM((B,), jnp.int32) for _ in range(NB)]
        + [pltpu.VMEM((D, B), jnp.float32) for _ in range(NB)]
        + [pltpu.SemaphoreType.DMA] * (2 * NB)
    ),
    compiler_params=pltpu.CompilerParams(needs_layout_passes=False),
)
def _emb_lookup(idx_hbm, tab_hbm, out_hbm, tab_v, idx_v0, idx_v1, out_v0,
                out_v1, sem_i0, sem_i1, sem_o0, sem_o1):
    idx_v = (idx_v0, idx_v1)
    out_v = (out_v0, out_v1)
    sem_i = (sem_i0, sem_i1)
    sem_o = (sem_o0, sem_o1)
    wid = lax.axis_index("s") * NC + lax.axis_index("c")
    pbase = wid * PPW

    pltpu.sync_copy(tab_hbm, tab_v)

    def idx_copy(p, b):
        return pltpu.make_async_copy(
            idx_hbm.at[pl.ds(p * B, B)], idx_v[b], sem_i[b]
        )

    def out_copy(p, b):
        s = p // C
        c = p - s * C
        return pltpu.make_async_copy(out_v[b], out_hbm.at[s, c], sem_o[b])

    def compute(b):
        def grp(g, carry):
            v = idx_v[b][pl.ds(g * L, L)] * D
            for d in range(D):
                out_v[b][d, pl.ds(g * L, L)] = plsc.load_gather(tab_v, [v + d])
            return carry

        lax.fori_loop(0, NGRP, grp, 0)

    for b in range(NB):
        idx_copy(pbase + b, b).start()
    for b in range(NB):
        idx_copy(pbase + b, b).wait()
        compute(b)
        out_copy(pbase + b, b).start()
        idx_copy(pbase + NB + b, b).start()

    def body(k, carry):
        for b in range(NB):
            p = pbase + (k + 1) * NB + b
            idx_copy(p, b).wait()
            out_copy(p - NB, b).wait()
            compute(b)
            out_copy(p, b).start()
            pnext = jnp.minimum(p + NB, pbase + PPW - 1)
            idx_copy(pnext, b).start()
        return carry

    lax.fori_loop(0, PPW // NB - 1, body, 0)

    for b in range(NB):
        idx_copy(pbase + PPW - 1, b).wait()
        out_copy(pbase + PPW - NB + b, b).wait()


def kernel(inputs, table):
    idx_t = jnp.transpose(inputs, (1, 2, 0)).reshape(N)
    tab_flat = table.reshape(V * D)
    out = _emb_lookup(idx_t, tab_flat)
    return jnp.transpose(out, (3, 0, 1, 2))


# speedup vs baseline: 31.7999x; 2.2308x over previous
import functools

import jax
import jax.numpy as jnp
from jax import lax
from jax.experimental import pallas as pl
from jax.experimental.pallas import tpu as pltpu
from jax.experimental.pallas import tpu_sc as plsc

B, S, C, D = 1024, 200, 16, 16
V = 1000
NP = S * C
N = NP * B

_info = plsc.get_sparse_core_info()
NC, NS, L = _info.num_cores, _info.num_subcores, _info.num_lanes
NW = NC * NS
PPW = NP // NW
NB = 2
NGRP = B // L

_mesh = plsc.VectorSubcoreMesh(core_axis_name="c", subcore_axis_name="s")


@functools.partial(
    pl.kernel,
    mesh=_mesh,
    out_type=jax.ShapeDtypeStruct((S, C, D, B), jnp.float32),
    scratch_types=(
        [pltpu.VMEM((V * D,), jnp.float32)]
        + [pltpu.VMEM((B,), jnp.int32) for _ in range(NB)]
        + [pltpu.VMEM((D, B), jnp.float32) for _ in range(NB)]
        + [pltpu.SemaphoreType.DMA] * (2 * NB)
    ),
    compiler_params=pltpu.CompilerParams(needs_layout_passes=False),
)
def _emb_lookup(idx_hbm, tab_hbm, out_hbm, tab_v, idx_v0, idx_v1, out_v0,
                out_v1, sem_i0, sem_i1, sem_o0, sem_o1):
    idx_v = (idx_v0, idx_v1)
    out_v = (out_v0, out_v1)
    sem_i = (sem_i0, sem_i1)
    sem_o = (sem_o0, sem_o1)
    wid = lax.axis_index("s") * NC + lax.axis_index("c")
    pbase = wid * PPW

    pltpu.sync_copy(tab_hbm, tab_v)

    def idx_copy(p, b):
        return pltpu.make_async_copy(
            idx_hbm.at[pl.ds(p * B, B)], idx_v[b], sem_i[b]
        )

    def out_copy(p, b):
        s = p // C
        c = p - s * C
        return pltpu.make_async_copy(out_v[b], out_hbm.at[s, c], sem_o[b])

    def compute(b):
        @plsc.parallel_loop(0, NGRP, unroll=4)
        def grp(g):
            v = idx_v[b][pl.ds(g * L, L)] * D
            for d in range(D):
                out_v[b][d, pl.ds(g * L, L)] = plsc.load_gather(tab_v, [v + d])

    for b in range(NB):
        idx_copy(pbase + b, b).start()
    for b in range(NB):
        idx_copy(pbase + b, b).wait()
        compute(b)
        out_copy(pbase + b, b).start()
        idx_copy(pbase + NB + b, b).start()

    def body(k, carry):
        for b in range(NB):
            p = pbase + (k + 1) * NB + b
            idx_copy(p, b).wait()
            out_copy(p - NB, b).wait()
            compute(b)
            out_copy(p, b).start()
            pnext = jnp.minimum(p + NB, pbase + PPW - 1)
            idx_copy(pnext, b).start()
        return carry

    lax.fori_loop(0, PPW // NB - 1, body, 0)

    for b in range(NB):
        idx_copy(pbase + PPW - 1, b).wait()
        out_copy(pbase + PPW - NB + b, b).wait()


def kernel(inputs, table):
    idx_t = jnp.transpose(inputs, (1, 2, 0)).reshape(N)
    tab_flat = table.reshape(V * D)
    out = _emb_lookup(idx_t, tab_flat)
    return jnp.transpose(out, (3, 0, 1, 2))


# speedup vs baseline: 93.1006x; 2.9277x over previous
import functools

import jax
import jax.numpy as jnp
from jax import lax
from jax.experimental import pallas as pl
from jax.experimental.pallas import tpu as pltpu
from jax.experimental.pallas import tpu_sc as plsc

B, S, C, D = 1024, 200, 16, 16
V = 1000
NP = S * C
N = NP * B

_info = plsc.get_sparse_core_info()
NC, NS, L = _info.num_cores, _info.num_subcores, _info.num_lanes
NW = NC * NS
PPW = NP // NW
NB = 2
NGRP = B // L

_mesh = plsc.VectorSubcoreMesh(core_axis_name="c", subcore_axis_name="s")


@functools.partial(
    pl.kernel,
    mesh=_mesh,
    out_type=jax.ShapeDtypeStruct((S, C, D, B), jnp.float32),
    scratch_types=(
        [pltpu.VMEM((V * (D + 1),), jnp.float32)]
        + [pltpu.VMEM((B,), jnp.int32) for _ in range(NB)]
        + [pltpu.VMEM((D, B), jnp.float32) for _ in range(NB)]
        + [pltpu.SemaphoreType.DMA] * (2 * NB)
    ),
    compiler_params=pltpu.CompilerParams(
        needs_layout_passes=False, disable_bounds_checks=True
    ),
)
def _emb_lookup(idx_hbm, tab_hbm, out_hbm, tab_v, idx_v0, idx_v1, out_v0,
                out_v1, sem_i0, sem_i1, sem_o0, sem_o1):
    idx_v = (idx_v0, idx_v1)
    out_v = (out_v0, out_v1)
    sem_i = (sem_i0, sem_i1)
    sem_o = (sem_o0, sem_o1)
    wid = lax.axis_index("s") * NC + lax.axis_index("c")
    pbase = wid * PPW

    pltpu.sync_copy(tab_hbm, tab_v)

    def idx_copy(p, b):
        return pltpu.make_async_copy(
            idx_hbm.at[pl.ds(p * B, B)], idx_v[b], sem_i[b]
        )

    def out_copy(p, b):
        s = p // C
        c = p - s * C
        return pltpu.make_async_copy(out_v[b], out_hbm.at[s, c], sem_o[b])

    def compute(b):
        @plsc.parallel_loop(0, NGRP, unroll=4)
        def grp(g):
            v = idx_v[b][pl.ds(g * L, L)] * (D + 1)
            for d in range(D):
                out_v[b][d, pl.ds(g * L, L)] = plsc.load_gather(tab_v, [v + d])

    for b in range(NB):
        idx_copy(pbase + b, b).start()
    for b in range(NB):
        idx_copy(pbase + b, b).wait()
        compute(b)
        out_copy(pbase + b, b).start()
        idx_copy(pbase + NB + b, b).start()

    def body(k, carry):
        for b in range(NB):
            p = pbase + (k + 1) * NB + b
            idx_copy(p, b).wait()
            out_copy(p - NB, b).wait()
            compute(b)
            out_copy(p, b).start()
            pnext = jnp.minimum(p + NB, pbase + PPW - 1)
            idx_copy(pnext, b).start()
        return carry

    lax.fori_loop(0, PPW // NB - 1, body, 0)

    for b in range(NB):
        idx_copy(pbase + PPW - 1, b).wait()
        out_copy(pbase + PPW - NB + b, b).wait()


def kernel(inputs, table):
    idx_t = jnp.transpose(inputs, (1, 2, 0)).reshape(N)
    tab_flat = jnp.pad(table, ((0, 0), (0, 1))).reshape(V * (D + 1))
    out = _emb_lookup(idx_t, tab_flat)
    return jnp.transpose(out, (3, 0, 1, 2))
